# baseline (speedup 1.0000x reference)
import jax
import jax.numpy as jnp
from jax.experimental import pallas as pl
from jax.experimental.pallas import tpu as pltpu

_B = 16384
_BB = 512
_NUM_EMB = 256
_N_LEVELS = 4
_E_DIM = 64
_BETA = 0.25
_LOSS_SCALE = (1.0 + _BETA) / (_N_LEVELS * _B * _E_DIM)


def _dot(a, b, precision=None):
    return jax.lax.dot_general(
        a, b, (((1,), (0,)), ((), ())),
        precision=precision,
        preferred_element_type=jnp.float32)


def _rqvae_kernel(x_ref, ew0, eb0, ew1, eb1, ew2, eb2, ew3, eb3,
                  cbt_ref, cb_ref, cbsq_ref,
                  dw0, db0, dw1, db1, dw2, db2, dw3, db3,
                  out_ref, idx_ref, loss_ref,
                  s512, s256, s128, s64a, s64b):
    s512[:] = jnp.maximum(_dot(x_ref[:], ew0[:]) + eb0[:], 0.0)
    s256[:] = jnp.maximum(_dot(s512[:], ew1[:]) + eb1[:], 0.0)
    s128[:] = jnp.maximum(_dot(s256[:], ew2[:]) + eb2[:], 0.0)
    s64a[:] = _dot(s128[:], ew3[:]) + eb3[:]
    s64b[:] = jnp.zeros((_BB, _E_DIM), jnp.float32)

    k_iota = jax.lax.broadcasted_iota(jnp.int32, (_BB, _NUM_EMB), 1)
    sse = jnp.zeros((1, 1), jnp.float32)
    for l in range(_N_LEVELS):
        r = s64a[:]
        rsq = jnp.sum(r * r, axis=1, keepdims=True)
        d = (rsq + cbsq_ref[l]) - 2.0 * _dot(r, cbt_ref[l])
        m = jnp.min(d, axis=1, keepdims=True)
        idxm = jnp.min(jnp.where(d <= m, k_iota, _NUM_EMB), axis=1,
                       keepdims=True)
        onehot = (k_iota == idxm).astype(jnp.float32)
        xq = _dot(onehot, cb_ref[l], precision=jax.lax.Precision.HIGHEST)
        diff = xq - r
        sse = sse + jnp.sum(diff * diff).reshape(1, 1)
        s64a[:] = r - xq
        s64b[:] = s64b[:] + xq
        idx_ref[:, l:l + 1] = idxm

    contrib = sse * _LOSS_SCALE

    @pl.when(pl.program_id(0) == 0)
    def _init():
        loss_ref[:, :] = contrib

    @pl.when(pl.program_id(0) != 0)
    def _acc():
        loss_ref[:, :] = loss_ref[:, :] + contrib

    s128[:] = jnp.maximum(_dot(s64b[:], dw0[:]) + db0[:], 0.0)
    s256[:] = jnp.maximum(_dot(s128[:], dw1[:]) + db1[:], 0.0)
    s512[:] = jnp.maximum(_dot(s256[:], dw2[:]) + db2[:], 0.0)
    out_ref[:] = _dot(s512[:], dw3[:]) + db3[:]


def _full(shape):
    return pl.BlockSpec(shape, lambda i: (0,) * len(shape))


def kernel(x, enc_W0, enc_b0, enc_W1, enc_b1, enc_W2, enc_b2, enc_W3, enc_b3,
           codebooks, dec_W0, dec_b0, dec_W1, dec_b1, dec_W2, dec_b2,
           dec_W3, dec_b3):
    in_dim = x.shape[1]
    out_dim = dec_W3.shape[1]
    cbt = jnp.transpose(codebooks, (0, 2, 1))
    cbsq = jnp.sum(codebooks * codebooks, axis=2)[:, None, :]
    weights = (enc_W0, enc_b0, enc_W1, enc_b1, enc_W2, enc_b2, enc_W3, enc_b3,
               cbt, codebooks, cbsq,
               dec_W0, dec_b0, dec_W1, dec_b1, dec_W2, dec_b2, dec_W3, dec_b3)
    in_specs = [pl.BlockSpec((_BB, in_dim), lambda i: (i, 0))]
    in_specs += [_full(w.shape) for w in weights]
    out, idx, loss = pl.pallas_call(
        _rqvae_kernel,
        grid=(_B // _BB,),
        in_specs=in_specs,
        out_specs=[
            pl.BlockSpec((_BB, out_dim), lambda i: (i, 0)),
            pl.BlockSpec((_BB, _N_LEVELS), lambda i: (i, 0)),
            pl.BlockSpec((1, 1), lambda i: (0, 0)),
        ],
        out_shape=[
            jax.ShapeDtypeStruct((_B, out_dim), jnp.float32),
            jax.ShapeDtypeStruct((_B, _N_LEVELS), jnp.int32),
            jax.ShapeDtypeStruct((1, 1), jnp.float32),
        ],
        scratch_shapes=[
            pltpu.VMEM((_BB, 512), jnp.float32),
            pltpu.VMEM((_BB, 256), jnp.float32),
            pltpu.VMEM((_BB, 128), jnp.float32),
            pltpu.VMEM((_BB, _E_DIM), jnp.float32),
            pltpu.VMEM((_BB, _E_DIM), jnp.float32),
        ],
        compiler_params=pltpu.CompilerParams(
            dimension_semantics=("arbitrary",)),
    )(x, *weights)
    return out, loss[0, 0], idx


# speedup vs baseline: 1.3131x; 1.3131x over previous
import jax
import jax.numpy as jnp
from jax.experimental import pallas as pl
from jax.experimental.pallas import tpu as pltpu

_B = 16384
_BB = 512
_NUM_EMB = 256
_N_LEVELS = 4
_E_DIM = 64
_BETA = 0.25
_LOSS_SCALE = (1.0 + _BETA) / (_N_LEVELS * _B * _E_DIM)


def _dot(a, b, precision=None):
    return jax.lax.dot_general(
        a, b, (((1,), (0,)), ((), ())),
        precision=precision,
        preferred_element_type=jnp.float32)


def _doti(a, b):
    return jax.lax.dot_general(
        a, b, (((1,), (0,)), ((), ())),
        preferred_element_type=jnp.int32)


def _rqvae_kernel(x_ref, ew0, eb0, ew1, eb1, ew2, eb2, ew3, eb3,
                  cbt_ref, p0_ref, p1_ref, p2_ref, p3_ref, cbsq_ref,
                  dw0, db0, dw1, db1, dw2, db2, dw3, db3,
                  out_ref, idx_ref, loss_ref,
                  s512, s256, s128, s64a, s64b):
    s512[:] = jnp.maximum(_dot(x_ref[:], ew0[:]) + eb0[:], 0.0)
    s256[:] = jnp.maximum(_dot(s512[:], ew1[:]) + eb1[:], 0.0)
    s128[:] = jnp.maximum(_dot(s256[:], ew2[:]) + eb2[:], 0.0)
    s64a[:] = _dot(s128[:], ew3[:]) + eb3[:]
    s64b[:] = jnp.zeros((_BB, _E_DIM), jnp.float32)

    k_iota = jax.lax.broadcasted_iota(jnp.int32, (_BB, _NUM_EMB), 1)
    sse = jnp.zeros((1, 1), jnp.float32)
    for l in range(_N_LEVELS):
        r = s64a[:]
        rsq = jnp.sum(r * r, axis=1, keepdims=True)
        d = (rsq + cbsq_ref[l]) - 2.0 * _dot(r, cbt_ref[l])
        m = jnp.min(d, axis=1, keepdims=True)
        idxm = jnp.min(jnp.where(d <= m, k_iota, _NUM_EMB), axis=1,
                       keepdims=True)
        oh8 = (k_iota == idxm).astype(jnp.int8)
        q0 = _doti(oh8, p0_ref[l])
        q1 = _doti(oh8, p1_ref[l])
        q2 = _doti(oh8, p2_ref[l])
        q3 = _doti(oh8, p3_ref[l])
        xq_bits = ((q0 & 0xFF) | ((q1 & 0xFF) << 8)
                   | ((q2 & 0xFF) << 16) | ((q3 & 0xFF) << 24))
        xq = jax.lax.bitcast_convert_type(xq_bits, jnp.float32)
        diff = xq - r
        sse = sse + jnp.sum(diff * diff).reshape(1, 1)
        s64a[:] = r - xq
        s64b[:] = s64b[:] + xq
        idx_ref[:, l:l + 1] = idxm

    contrib = sse * _LOSS_SCALE

    @pl.when(pl.program_id(0) == 0)
    def _init():
        loss_ref[:, :] = contrib

    @pl.when(pl.program_id(0) != 0)
    def _acc():
        loss_ref[:, :] = loss_ref[:, :] + contrib

    s128[:] = jnp.maximum(_dot(s64b[:], dw0[:]) + db0[:], 0.0)
    s256[:] = jnp.maximum(_dot(s128[:], dw1[:]) + db1[:], 0.0)
    s512[:] = jnp.maximum(_dot(s256[:], dw2[:]) + db2[:], 0.0)
    out_ref[:] = _dot(s512[:], dw3[:]) + db3[:]


def _full(shape):
    return pl.BlockSpec(shape, lambda i: (0,) * len(shape))


def kernel(x, enc_W0, enc_b0, enc_W1, enc_b1, enc_W2, enc_b2, enc_W3, enc_b3,
           codebooks, dec_W0, dec_b0, dec_W1, dec_b1, dec_W2, dec_b2,
           dec_W3, dec_b3):
    in_dim = x.shape[1]
    out_dim = dec_W3.shape[1]
    cbt = jnp.transpose(codebooks, (0, 2, 1))
    cbsq = jnp.sum(codebooks * codebooks, axis=2)[:, None, :]
    bits = jax.lax.bitcast_convert_type(codebooks, jnp.uint32)
    p0 = (bits & 0xFF).astype(jnp.int8)
    p1 = ((bits >> 8) & 0xFF).astype(jnp.int8)
    p2 = ((bits >> 16) & 0xFF).astype(jnp.int8)
    p3 = ((bits >> 24) & 0xFF).astype(jnp.int8)
    weights = (enc_W0, enc_b0, enc_W1, enc_b1, enc_W2, enc_b2, enc_W3, enc_b3,
               cbt, p0, p1, p2, p3, cbsq,
               dec_W0, dec_b0, dec_W1, dec_b1, dec_W2, dec_b2, dec_W3, dec_b3)
    in_specs = [pl.BlockSpec((_BB, in_dim), lambda i: (i, 0))]
    in_specs += [_full(w.shape) for w in weights]
    out, idx, loss = pl.pallas_call(
        _rqvae_kernel,
        grid=(_B // _BB,),
        in_specs=in_specs,
        out_specs=[
            pl.BlockSpec((_BB, out_dim), lambda i: (i, 0)),
            pl.BlockSpec((_BB, _N_LEVELS), lambda i: (i, 0)),
            pl.BlockSpec((1, 1), lambda i: (0, 0)),
        ],
        out_shape=[
            jax.ShapeDtypeStruct((_B, out_dim), jnp.float32),
            jax.ShapeDtypeStruct((_B, _N_LEVELS), jnp.int32),
            jax.ShapeDtypeStruct((1, 1), jnp.float32),
        ],
        scratch_shapes=[
            pltpu.VMEM((_BB, 512), jnp.float32),
            pltpu.VMEM((_BB, 256), jnp.float32),
            pltpu.VMEM((_BB, 128), jnp.float32),
            pltpu.VMEM((_BB, _E_DIM), jnp.float32),
            pltpu.VMEM((_BB, _E_DIM), jnp.float32),
        ],
        compiler_params=pltpu.CompilerParams(
            dimension_semantics=("arbitrary",)),
    )(x, *weights)
    return out, loss[0, 0], idx


# speedup vs baseline: 1.6559x; 1.2610x over previous
import jax
import jax.numpy as jnp
from jax.experimental import pallas as pl
from jax.experimental.pallas import tpu as pltpu

_B = 16384
_BB = 1024
_NUM_EMB = 256
_N_LEVELS = 4
_E_DIM = 64
_BETA = 0.25
_LOSS_SCALE = (1.0 + _BETA) / (_N_LEVELS * _B * _E_DIM)


def _dot(a, b, precision=None):
    return jax.lax.dot_general(
        a, b, (((1,), (0,)), ((), ())),
        precision=precision,
        preferred_element_type=jnp.float32)


def _doti(a, b):
    return jax.lax.dot_general(
        a, b, (((1,), (0,)), ((), ())),
        preferred_element_type=jnp.int32)


def _rqvae_kernel(x_ref, ew0, eb0, ew1, eb1, ew2, eb2, ew3, eb3,
                  cbt_ref, p0_ref, p1_ref, p2_ref, p3_ref, cbsq_ref,
                  dw0, db0, dw1, db1, dw2, db2, dw3, db3,
                  out_ref, idx_ref, loss_ref,
                  s512, s256, s128, s64a, s64b):
    s512[:] = jnp.maximum(_dot(x_ref[:], ew0[:]) + eb0[:], 0.0)
    s256[:] = jnp.maximum(_dot(s512[:], ew1[:]) + eb1[:], 0.0)
    s128[:] = jnp.maximum(_dot(s256[:], ew2[:]) + eb2[:], 0.0)
    s64a[:] = _dot(s128[:], ew3[:]) + eb3[:]
    s64b[:] = jnp.zeros((_BB, _E_DIM), jnp.float32)

    k_iota = jax.lax.broadcasted_iota(jnp.int32, (_BB, _NUM_EMB), 1)
    sse = jnp.zeros((1, 1), jnp.float32)
    for l in range(_N_LEVELS):
        r = s64a[:]
        rsq = jnp.sum(r * r, axis=1, keepdims=True)
        d = (rsq + cbsq_ref[l]) + _dot(r, cbt_ref[l])
        m = jnp.min(d, axis=1, keepdims=True)
        idxm = jnp.min(jnp.where(d <= m, k_iota, _NUM_EMB), axis=1,
                       keepdims=True)
        oh8 = (k_iota == idxm).astype(jnp.int8)
        q0 = _doti(oh8, p0_ref[l])
        q1 = _doti(oh8, p1_ref[l])
        q2 = _doti(oh8, p2_ref[l])
        q3 = _doti(oh8, p3_ref[l])
        xq_bits = ((q0 & 0xFF) | ((q1 & 0xFF) << 8)
                   | ((q2 & 0xFF) << 16) | ((q3 & 0xFF) << 24))
        xq = jax.lax.bitcast_convert_type(xq_bits, jnp.float32)
        diff = xq - r
        sse = sse + jnp.sum(diff * diff).reshape(1, 1)
        s64a[:] = r - xq
        s64b[:] = s64b[:] + xq
        idx_ref[:, l:l + 1] = idxm

    contrib = sse * _LOSS_SCALE

    @pl.when(pl.program_id(0) == 0)
    def _init():
        loss_ref[:, :] = contrib

    @pl.when(pl.program_id(0) != 0)
    def _acc():
        loss_ref[:, :] = loss_ref[:, :] + contrib

    s128[:] = jnp.maximum(_dot(s64b[:], dw0[:]) + db0[:], 0.0)
    s256[:] = jnp.maximum(_dot(s128[:], dw1[:]) + db1[:], 0.0)
    s512[:] = jnp.maximum(_dot(s256[:], dw2[:]) + db2[:], 0.0)
    out_ref[:] = _dot(s512[:], dw3[:]) + db3[:]


def _full(shape):
    return pl.BlockSpec(shape, lambda i: (0,) * len(shape))


def kernel(x, enc_W0, enc_b0, enc_W1, enc_b1, enc_W2, enc_b2, enc_W3, enc_b3,
           codebooks, dec_W0, dec_b0, dec_W1, dec_b1, dec_W2, dec_b2,
           dec_W3, dec_b3):
    in_dim = x.shape[1]
    out_dim = dec_W3.shape[1]
    cbt = -2.0 * jnp.transpose(codebooks, (0, 2, 1))
    cbsq = jnp.sum(codebooks * codebooks, axis=2)[:, None, :]
    bits = jax.lax.bitcast_convert_type(codebooks, jnp.uint32)
    p0 = (bits & 0xFF).astype(jnp.int8)
    p1 = ((bits >> 8) & 0xFF).astype(jnp.int8)
    p2 = ((bits >> 16) & 0xFF).astype(jnp.int8)
    p3 = ((bits >> 24) & 0xFF).astype(jnp.int8)
    weights = (enc_W0, enc_b0, enc_W1, enc_b1, enc_W2, enc_b2, enc_W3, enc_b3,
               cbt, p0, p1, p2, p3, cbsq,
               dec_W0, dec_b0, dec_W1, dec_b1, dec_W2, dec_b2, dec_W3, dec_b3)
    in_specs = [pl.BlockSpec((_BB, in_dim), lambda i: (i, 0))]
    in_specs += [_full(w.shape) for w in weights]
    out, idx, loss = pl.pallas_call(
        _rqvae_kernel,
        grid=(_B // _BB,),
        in_specs=in_specs,
        out_specs=[
            pl.BlockSpec((_BB, out_dim), lambda i: (i, 0)),
            pl.BlockSpec((_BB, _N_LEVELS), lambda i: (i, 0)),
            pl.BlockSpec((1, 1), lambda i: (0, 0)),
        ],
        out_shape=[
            jax.ShapeDtypeStruct((_B, out_dim), jnp.float32),
            jax.ShapeDtypeStruct((_B, _N_LEVELS), jnp.int32),
            jax.ShapeDtypeStruct((1, 1), jnp.float32),
        ],
        scratch_shapes=[
            pltpu.VMEM((_BB, 512), jnp.float32),
            pltpu.VMEM((_BB, 256), jnp.float32),
            pltpu.VMEM((_BB, 128), jnp.float32),
            pltpu.VMEM((_BB, _E_DIM), jnp.float32),
            pltpu.VMEM((_BB, _E_DIM), jnp.float32),
        ],
        compiler_params=pltpu.CompilerParams(
            dimension_semantics=("arbitrary",)),
    )(x, *weights)
    return out, loss[0, 0], idx


# speedup vs baseline: 1.8013x; 1.0878x over previous
import jax
import jax.numpy as jnp
from jax.experimental import pallas as pl
from jax.experimental.pallas import tpu as pltpu

_B = 16384
_BB = 2048
_NUM_EMB = 256
_N_LEVELS = 4
_E_DIM = 64
_BETA = 0.25
_LOSS_SCALE = (1.0 + _BETA) / (_N_LEVELS * _B * _E_DIM)


def _dot(a, b, precision=None):
    return jax.lax.dot_general(
        a, b, (((1,), (0,)), ((), ())),
        precision=precision,
        preferred_element_type=jnp.float32)


def _doti(a, b):
    return jax.lax.dot_general(
        a, b, (((1,), (0,)), ((), ())),
        preferred_element_type=jnp.int32)


def _rqvae_kernel(x_ref, ew0, eb0, ew1, eb1, ew2, eb2, ew3, eb3,
                  cbt_ref, p0_ref, p1_ref, p2_ref, p3_ref, cbsq_ref,
                  dw0, db0, dw1, db1, dw2, db2, dw3, db3,
                  out_ref, idx_ref, loss_ref,
                  s512, s256, s128, s64a, s64b):
    s512[:] = jnp.maximum(_dot(x_ref[:], ew0[:]) + eb0[:], 0.0)
    s256[:] = jnp.maximum(_dot(s512[:], ew1[:]) + eb1[:], 0.0)
    s128[:] = jnp.maximum(_dot(s256[:], ew2[:]) + eb2[:], 0.0)
    s64a[:] = _dot(s128[:], ew3[:]) + eb3[:]
    s64b[:] = jnp.zeros((_BB, _E_DIM), jnp.float32)

    k_iota = jax.lax.broadcasted_iota(jnp.int32, (_BB, _NUM_EMB), 1)
    sse = jnp.zeros((1, 1), jnp.float32)
    for l in range(_N_LEVELS):
        r = s64a[:]
        rsq = jnp.sum(r * r, axis=1, keepdims=True)
        d = (rsq + cbsq_ref[l]) + _dot(r, cbt_ref[l])
        m = jnp.min(d, axis=1, keepdims=True)
        idxm = jnp.min(jnp.where(d <= m, k_iota, _NUM_EMB), axis=1,
                       keepdims=True)
        oh8 = (k_iota == idxm).astype(jnp.int8)
        q0 = _doti(oh8, p0_ref[l])
        q1 = _doti(oh8, p1_ref[l])
        q2 = _doti(oh8, p2_ref[l])
        q3 = _doti(oh8, p3_ref[l])
        xq_bits = ((q0 & 0xFF) | ((q1 & 0xFF) << 8)
                   | ((q2 & 0xFF) << 16) | ((q3 & 0xFF) << 24))
        xq = jax.lax.bitcast_convert_type(xq_bits, jnp.float32)
        diff = xq - r
        sse = sse + jnp.sum(diff * diff).reshape(1, 1)
        s64a[:] = r - xq
        s64b[:] = s64b[:] + xq
        idx_ref[:, l:l + 1] = idxm

    contrib = sse * _LOSS_SCALE

    @pl.when(pl.program_id(0) == 0)
    def _init():
        loss_ref[:, :] = contrib

    @pl.when(pl.program_id(0) != 0)
    def _acc():
        loss_ref[:, :] = loss_ref[:, :] + contrib

    s128[:] = jnp.maximum(_dot(s64b[:], dw0[:]) + db0[:], 0.0)
    s256[:] = jnp.maximum(_dot(s128[:], dw1[:]) + db1[:], 0.0)
    s512[:] = jnp.maximum(_dot(s256[:], dw2[:]) + db2[:], 0.0)
    out_ref[:] = _dot(s512[:], dw3[:]) + db3[:]


def _full(shape):
    return pl.BlockSpec(shape, lambda i: (0,) * len(shape))


def kernel(x, enc_W0, enc_b0, enc_W1, enc_b1, enc_W2, enc_b2, enc_W3, enc_b3,
           codebooks, dec_W0, dec_b0, dec_W1, dec_b1, dec_W2, dec_b2,
           dec_W3, dec_b3):
    in_dim = x.shape[1]
    out_dim = dec_W3.shape[1]
    cbt = -2.0 * jnp.transpose(codebooks, (0, 2, 1))
    cbsq = jnp.sum(codebooks * codebooks, axis=2)[:, None, :]
    bits = jax.lax.bitcast_convert_type(codebooks, jnp.uint32)
    p0 = (bits & 0xFF).astype(jnp.int8)
    p1 = ((bits >> 8) & 0xFF).astype(jnp.int8)
    p2 = ((bits >> 16) & 0xFF).astype(jnp.int8)
    p3 = ((bits >> 24) & 0xFF).astype(jnp.int8)
    weights = (enc_W0, enc_b0, enc_W1, enc_b1, enc_W2, enc_b2, enc_W3, enc_b3,
               cbt, p0, p1, p2, p3, cbsq,
               dec_W0, dec_b0, dec_W1, dec_b1, dec_W2, dec_b2, dec_W3, dec_b3)
    in_specs = [pl.BlockSpec((_BB, in_dim), lambda i: (i, 0))]
    in_specs += [_full(w.shape) for w in weights]
    out, idx, loss = pl.pallas_call(
        _rqvae_kernel,
        grid=(_B // _BB,),
        in_specs=in_specs,
        out_specs=[
            pl.BlockSpec((_BB, out_dim), lambda i: (i, 0)),
            pl.BlockSpec((_BB, _N_LEVELS), lambda i: (i, 0)),
            pl.BlockSpec((1, 1), lambda i: (0, 0)),
        ],
        out_shape=[
            jax.ShapeDtypeStruct((_B, out_dim), jnp.float32),
            jax.ShapeDtypeStruct((_B, _N_LEVELS), jnp.int32),
            jax.ShapeDtypeStruct((1, 1), jnp.float32),
        ],
        scratch_shapes=[
            pltpu.VMEM((_BB, 512), jnp.float32),
            pltpu.VMEM((_BB, 256), jnp.float32),
            pltpu.VMEM((_BB, 128), jnp.float32),
            pltpu.VMEM((_BB, _E_DIM), jnp.float32),
            pltpu.VMEM((_BB, _E_DIM), jnp.float32),
        ],
        compiler_params=pltpu.CompilerParams(
            dimension_semantics=("arbitrary",)),
    )(x, *weights)
    return out, loss[0, 0], idx


# speedup vs baseline: 1.8099x; 1.0048x over previous
import jax
import jax.numpy as jnp
from jax.experimental import pallas as pl
from jax.experimental.pallas import tpu as pltpu

_B = 16384
_BB = 2048
_NUM_EMB = 256
_N_LEVELS = 4
_E_DIM = 64
_BETA = 0.25
_LOSS_SCALE = (1.0 + _BETA) / (_N_LEVELS * _B * _E_DIM)


def _dot(a, b, precision=None):
    return jax.lax.dot_general(
        a, b, (((1,), (0,)), ((), ())),
        precision=precision,
        preferred_element_type=jnp.float32)


def _doti(a, b):
    return jax.lax.dot_general(
        a, b, (((1,), (0,)), ((), ())),
        preferred_element_type=jnp.int32)


def _rqvae_kernel(x_ref, ew0, eb0, ew1, eb1, ew2, eb2, ew3, eb3,
                  cbt_ref, p0_ref, p1_ref, p2_ref, p3_ref, cbsq_ref,
                  dw0, db0, dw1, db1, dw2, db2, dw3, db3,
                  out_ref, idx_ref, loss_ref,
                  s512, s256, s128, s64a, s64b):
    s512[:] = jnp.maximum(_dot(x_ref[:], ew0[:]) + eb0[:], 0.0)
    s256[:] = jnp.maximum(_dot(s512[:], ew1[:]) + eb1[:], 0.0)
    s128[:] = jnp.maximum(_dot(s256[:], ew2[:]) + eb2[:], 0.0)
    s64a[:] = _dot(s128[:], ew3[:]) + eb3[:]
    s64b[:] = jnp.zeros((_BB, _E_DIM), jnp.float32)

    k_iota = jax.lax.broadcasted_iota(jnp.int32, (_BB, _NUM_EMB), 1)
    sse = jnp.zeros((1, 1), jnp.float32)
    for l in range(_N_LEVELS):
        r = s64a[:]
        rsq = jnp.sum(r * r, axis=1, keepdims=True)
        d = (rsq + cbsq_ref[l]) + _dot(r, cbt_ref[l])
        m = jnp.min(d, axis=1, keepdims=True)
        idxm = jnp.min(jnp.where(d <= m, k_iota, _NUM_EMB), axis=1,
                       keepdims=True)
        oh8 = (k_iota == idxm).astype(jnp.int8)
        q0 = _doti(oh8, p0_ref[l])
        q1 = _doti(oh8, p1_ref[l])
        q2 = _doti(oh8, p2_ref[l])
        q3 = _doti(oh8, p3_ref[l])
        xq_bits = ((q0 & 0xFF) | ((q1 & 0xFF) << 8)
                   | ((q2 & 0xFF) << 16) | (q3 << 24))
        xq = jax.lax.bitcast_convert_type(xq_bits, jnp.float32)
        diff = xq - r
        sse = sse + jnp.sum(diff * diff).reshape(1, 1)
        s64a[:] = r - xq
        s64b[:] = s64b[:] + xq
        idx_ref[:, l:l + 1] = idxm

    contrib = sse * _LOSS_SCALE

    @pl.when(pl.program_id(0) == 0)
    def _init():
        loss_ref[:, :] = contrib

    @pl.when(pl.program_id(0) != 0)
    def _acc():
        loss_ref[:, :] = loss_ref[:, :] + contrib

    s128[:] = jnp.maximum(_dot(s64b[:], dw0[:]) + db0[:], 0.0)
    s256[:] = jnp.maximum(_dot(s128[:], dw1[:]) + db1[:], 0.0)
    s512[:] = jnp.maximum(_dot(s256[:], dw2[:]) + db2[:], 0.0)
    out_ref[:] = _dot(s512[:], dw3[:]) + db3[:]


def _full(shape):
    return pl.BlockSpec(shape, lambda i: (0,) * len(shape))


def kernel(x, enc_W0, enc_b0, enc_W1, enc_b1, enc_W2, enc_b2, enc_W3, enc_b3,
           codebooks, dec_W0, dec_b0, dec_W1, dec_b1, dec_W2, dec_b2,
           dec_W3, dec_b3):
    in_dim = x.shape[1]
    out_dim = dec_W3.shape[1]
    cbt = -2.0 * jnp.transpose(codebooks, (0, 2, 1))
    cbsq = jnp.sum(codebooks * codebooks, axis=2)[:, None, :]
    bits = jax.lax.bitcast_convert_type(codebooks, jnp.uint32)
    p0 = (bits & 0xFF).astype(jnp.int8)
    p1 = ((bits >> 8) & 0xFF).astype(jnp.int8)
    p2 = ((bits >> 16) & 0xFF).astype(jnp.int8)
    p3 = ((bits >> 24) & 0xFF).astype(jnp.int8)
    weights = (enc_W0, enc_b0, enc_W1, enc_b1, enc_W2, enc_b2, enc_W3, enc_b3,
               cbt, p0, p1, p2, p3, cbsq,
               dec_W0, dec_b0, dec_W1, dec_b1, dec_W2, dec_b2, dec_W3, dec_b3)
    in_specs = [pl.BlockSpec((_BB, in_dim), lambda i: (i, 0))]
    in_specs += [_full(w.shape) for w in weights]
    out, idx, loss = pl.pallas_call(
        _rqvae_kernel,
        grid=(_B // _BB,),
        in_specs=in_specs,
        out_specs=[
            pl.BlockSpec((_BB, out_dim), lambda i: (i, 0)),
            pl.BlockSpec((_BB, _N_LEVELS), lambda i: (i, 0)),
            pl.BlockSpec((1, 1), lambda i: (0, 0)),
        ],
        out_shape=[
            jax.ShapeDtypeStruct((_B, out_dim), jnp.float32),
            jax.ShapeDtypeStruct((_B, _N_LEVELS), jnp.int32),
            jax.ShapeDtypeStruct((1, 1), jnp.float32),
        ],
        scratch_shapes=[
            pltpu.VMEM((_BB, 512), jnp.float32),
            pltpu.VMEM((_BB, 256), jnp.float32),
            pltpu.VMEM((_BB, 128), jnp.float32),
            pltpu.VMEM((_BB, _E_DIM), jnp.float32),
            pltpu.VMEM((_BB, _E_DIM), jnp.float32),
        ],
        compiler_params=pltpu.CompilerParams(
            dimension_semantics=("arbitrary",)),
    )(x, *weights)
    return out, loss[0, 0], idx
